# baseline (speedup 1.0000x reference)
import functools

import numpy as np
from itertools import combinations

import jax
import jax.numpy as jnp
from jax import lax
from jax.experimental import pallas as pl
from jax.experimental.pallas import tpu as pltpu
from jax.experimental.pallas import tpu_sc as plsc

F = 26
E = 16
FE = F * E
B = 4096
V = 1000000

_PAIRS = list(combinations(range(F), 2))
_LEFT = np.array([p[0] for p in _PAIRS], dtype=np.int32)
_RIGHT = np.array([p[1] for p in _PAIRS], dtype=np.int32)
_NP = len(_PAIRS)

_PSCAT = np.zeros((_NP, F * F), np.float32)
_PSCAT[np.arange(_NP), _LEFT * F + _RIGHT] = 1.0
_EYE_E = np.eye(E, dtype=np.float32)
_Q0 = np.einsum('ij,ef->iejf', np.triu(np.ones((F, F), np.float32), 1),
                _EYE_E).reshape(FE, FE)


_NC, _NS = 2, 16
_NW = _NC * _NS
_BPW = B // _NW
_IPW = _BPW * F
_CH = 128
_NCHUNK = _IPW // _CH


def _sc_gather_body(idx_hbm, t0, t1, t2, t3, w16, u_out, wsel_out,
                    idx_v, rows_v, wsel_v, sem, wsem):
    wid = lax.axis_index("s") * _NC + lax.axis_index("c")
    base = wid * _IPW
    pltpu.sync_copy(idx_hbm.at[wid], idx_v)

    def wbody(j, carry):
        pltpu.async_copy(w16.at[idx_v.at[j]],
                         wsel_v.at[pl.ds(j * _CH, _CH)], wsem)
        return carry
    lax.fori_loop(0, _NCHUNK, wbody, 0)

    jobs = [(t0, 0), (t1, 1), (t2, 2), (t3, 3)]
    for tref, slot in jobs:
        def tbody(j, carry, tref=tref):
            pltpu.async_copy(tref.at[idx_v.at[j]],
                             rows_v.at[pl.ds(j * _CH, _CH)], sem)
            return carry
        lax.fori_loop(0, _NCHUNK, tbody, 0)
        def twait(j, carry, tref=tref):
            pltpu.make_async_copy(tref.at[idx_v.at[j]],
                                  rows_v.at[pl.ds(j * _CH, _CH)], sem).wait()
            return carry
        lax.fori_loop(0, _NCHUNK, twait, 0)
        pltpu.sync_copy(rows_v, u_out.at[slot, pl.ds(base, _IPW)])

    def wwait(j, carry):
        pltpu.make_async_copy(w16.at[idx_v.at[j]],
                              wsel_v.at[pl.ds(j * _CH, _CH)], wsem).wait()
        return carry
    lax.fori_loop(0, _NCHUNK, wwait, 0)
    pltpu.sync_copy(wsel_v, wsel_out.at[pl.ds(base, _IPW)])


@functools.cache
def _sc_gather():
    return pl.kernel(
        _sc_gather_body,
        mesh=plsc.VectorSubcoreMesh(core_axis_name="c", subcore_axis_name="s",
                                    num_cores=_NC, num_subcores=_NS),
        compiler_params=pltpu.CompilerParams(use_tc_tiling_on_sc=False),
        out_type=[
            jax.ShapeDtypeStruct((4, B * F, E), jnp.float32),
            jax.ShapeDtypeStruct((B * F,), jnp.float32),
        ],
        scratch_types=[
            pltpu.VMEM((_NCHUNK, _CH), jnp.int32),
            pltpu.VMEM((_IPW, E), jnp.float32),
            pltpu.VMEM((_IPW,), jnp.float32),
            pltpu.SemaphoreType.DMA,
            pltpu.SemaphoreType.DMA,
        ],
    )


_BT = 1024

def _tc_body(xw_ref, u_ref, q_ref, b_ref, o_ref):
    acc = jnp.sum(xw_ref[...], axis=1, keepdims=True) + b_ref[0, 0]
    for t in range(4):
        u = u_ref[t]
        y = lax.dot_general(u, q_ref[t], (((1,), (0,)), ((), ())),
                            precision=lax.Precision.HIGHEST,
                            preferred_element_type=jnp.float32)
        acc = acc + jnp.sum(y * u, axis=1, keepdims=True)
    o_ref[...] = acc


def kernel(inputs, w, b, v0, v1, v2, v3, kernel_vec, kernel_num, kernel_mat):
    idx32 = inputs.astype(jnp.int32)
    idx = idx32.reshape(_NW, _NCHUNK, _CH)

    u_flat, xw_flat = _sc_gather()(idx, v0, v1, v2, v3, w)
    u = u_flat.reshape(4, B, FE)
    xw = xw_flat.reshape(B, F)

    a1 = (kernel_vec[0].T @ _PSCAT).reshape(E, F, F)
    q1 = jnp.einsum('eij,ef->iejf', a1, _EYE_E).reshape(FE, FE)
    a2 = (kernel_num[0, :, 0] @ _PSCAT).reshape(F, F)
    q2 = jnp.einsum('ij,ef->iejf', a2, _EYE_E).reshape(FE, FE)
    a3 = (_PSCAT.T @ kernel_mat.reshape(_NP, E * E)).reshape(F, F, E, E)
    q3 = a3.transpose(0, 2, 1, 3).reshape(FE, FE)
    qs = jnp.stack([jnp.asarray(_Q0), q1, q2, q3])

    logits = pl.pallas_call(
        _tc_body,
        grid=(B // _BT,),
        out_shape=jax.ShapeDtypeStruct((B, 1), jnp.float32),
        in_specs=[
            pl.BlockSpec((_BT, F), lambda i: (i, 0)),
            pl.BlockSpec((4, _BT, FE), lambda i: (0, i, 0)),
            pl.BlockSpec((4, FE, FE), lambda i: (0, 0, 0)),
            pl.BlockSpec(memory_space=pltpu.SMEM),
        ],
        out_specs=pl.BlockSpec((_BT, 1), lambda i: (i, 0)),
    )(xw, u, qs, b.reshape(1, 1))
    return logits
